# baseline (speedup 1.0000x reference)
import functools

import numpy as np
import jax
import jax.numpy as jnp
from jax import lax
from jax.experimental import pallas as pl
from jax.experimental.pallas import tpu as pltpu
from jax.experimental.pallas import tpu_sc as plsc

_F = 26
_V = 100000
_D = 16
_B = 4096
_NP = _F * (_F - 1) // 2

_IIN = np.array([i for i in range(_F) for _ in range(i)], dtype=np.int32)
_JJN = np.array([j for i in range(_F) for j in range(i)], dtype=np.int32)
_SI = np.zeros((_NP, _F), np.float32)
_SI[np.arange(_NP), _IIN] = 1.0
_SJ = np.zeros((_NP, _F), np.float32)
_SJ[np.arange(_NP), _JJN] = 1.0

_NC, _NS = 2, 16
_NW = _NC * _NS
_BPW = _B // _NW
_RPW = _BPW * _F
_CW = _BPW * _D


_TSUB = 16
_RSUB = _TSUB * _F
_NSUB = _BPW // _TSUB


def _sc_gather_t(tables3d, flat_x):
    mesh = plsc.VectorSubcoreMesh(
        core_axis_name="c", subcore_axis_name="s",
        num_cores=_NC, num_subcores=_NS)

    @functools.partial(
        pl.kernel,
        out_type=jax.ShapeDtypeStruct((_F, _NW * _CW), jnp.float32),
        mesh=mesh,
        scratch_types=[
            pltpu.VMEM((_RPW,), jnp.int32),
            pltpu.VMEM((_RSUB, _D), jnp.float32),
            pltpu.VMEM((_F * _CW,), jnp.float32),
            pltpu.SemaphoreType.DMA,
        ],
        compiler_params=pltpu.CompilerParams(use_tc_tiling_on_sc=True,
                                             needs_layout_passes=False),
    )
    def gather_k(table_hbm, idx_hbm, out_hbm, idx_v, rows_v, t_v, sem):
        wid = lax.axis_index("s") * _NC + lax.axis_index("c")
        base = wid * _RPW
        pltpu.sync_copy(idx_hbm.at[pl.ds(base, _RPW)], idx_v)

        dstep = lax.broadcasted_iota(jnp.int32, (_D,), 0) * _BPW

        def s_body(s, carry):
            ivs = [idx_v[pl.ds(s * _RSUB + k * _D, _D)]
                   for k in range(_RSUB // _D)]
            cps = [pltpu.async_copy(table_hbm.at[(k * _D + j) % _F,
                                                 ivs[k][j]],
                                    rows_v.at[k * _D + j], sem)
                   for k in range(_RSUB // _D) for j in range(_D)]
            for cp in cps:
                cp.wait()

            def t_body(tt, c2):
                t = s * _TSUB + tt
                for f in range(_F):
                    val = rows_v[tt * _F + f, :]
                    plsc.store_scatter(t_v, [dstep + (f * _CW + t)], val)
                return c2

            lax.fori_loop(0, _TSUB, t_body, 0)
            return carry

        lax.fori_loop(0, _NSUB, s_body, 0)
        for f in range(_F):
            pltpu.sync_copy(t_v.at[pl.ds(f * _CW, _CW)],
                            out_hbm.at[f, pl.ds(wid * _CW, _CW)])

    return gather_k(tables3d, flat_x)


_CH = 4
_COLS = _CH * _CW
_HI = dict(preferred_element_type=jnp.float32, precision=lax.Precision.HIGHEST)
_HX = dict(preferred_element_type=jnp.float32, precision=lax.Precision.HIGHEST)


def _tc_body(si_ref, sj_ref, sit_ref, sjt_ref, awt_ref, flag_ref, bias_ref,
             e_ref, out_ref):
    si = si_ref[...]
    sj = sj_ref[...]
    sit = sit_ref[...]
    sjt = sjt_ref[...]
    awt = awt_ref[...]
    flag = flag_ref[0]
    mx = jnp.max(awt, axis=0, keepdims=True)
    iota = lax.broadcasted_iota(jnp.int32, (5, _NP), 0)
    first = jnp.min(jnp.where(awt == mx, iota, 2**30), axis=0, keepdims=True)
    onehot = (iota == first).astype(jnp.float32)
    w = jnp.where(flag == 0, onehot, awt)
    car = w[0:1] + w[4:5] + 0.5 * (w[2:3] + w[3:4])
    cbr = w[1:2]
    ccr = 0.5 * (w[2:3] - w[3:4])

    eall = e_ref[...]
    dmat = jnp.dot(si - sj, eall, **_HI)
    s_abs = jnp.dot(ccr, jnp.abs(dmat), **_HI)
    bmat = (jnp.dot(sit * cbr, sj, **_HI)
            + jnp.dot(sjt * cbr, si, **_HI))
    m = jnp.dot(bmat, eall, **_HI)
    s_mult = 0.5 * jnp.sum(m * eall, axis=0, keepdims=True)
    arow = jnp.dot(car, si + sj, **_HX)
    s_lin = jnp.dot(arow, eall, **_HX)
    s = s_abs + s_mult + s_lin

    pieces = []
    for c in range(_CH):
        acc = s[:, c * _CW:c * _CW + _BPW]
        for d in range(1, _D):
            acc = acc + s[:, c * _CW + d * _BPW:c * _CW + (d + 1) * _BPW]
        pieces.append(acc)
    tot = jnp.concatenate(pieces, axis=1) + bias_ref[0]
    out_ref[...] = 1.0 / (1.0 + jnp.exp(-tot))


def kernel(x, flag, tables, arch_weights, bias):
    x = x.astype(jnp.int32)
    flat_x = x.reshape(_B * _F)
    e2 = _sc_gather_t(tables, flat_x)
    flag_arr = jnp.asarray(flag, jnp.int32).reshape(1)
    out2d = pl.pallas_call(
        _tc_body,
        grid=(_NW // _CH,),
        in_specs=[
            pl.BlockSpec((_NP, _F), lambda i: (0, 0)),
            pl.BlockSpec((_NP, _F), lambda i: (0, 0)),
            pl.BlockSpec((_F, _NP), lambda i: (0, 0)),
            pl.BlockSpec((_F, _NP), lambda i: (0, 0)),
            pl.BlockSpec((5, _NP), lambda i: (0, 0)),
            pl.BlockSpec(memory_space=pltpu.SMEM),
            pl.BlockSpec(memory_space=pltpu.SMEM),
            pl.BlockSpec((_F, _COLS), lambda i: (0, i)),
        ],
        out_specs=pl.BlockSpec((1, _CH * _BPW), lambda i: (0, i)),
        out_shape=jax.ShapeDtypeStruct((1, _B), jnp.float32),
    )(jnp.asarray(_SI), jnp.asarray(_SJ), jnp.asarray(_SI.T), jnp.asarray(_SJ.T),
      arch_weights.T, flag_arr, bias, e2)
    return out2d.reshape(_B)


# speedup vs baseline: 3.1685x; 3.1685x over previous
import functools

import numpy as np
import jax
import jax.numpy as jnp
from jax import lax
from jax.experimental import pallas as pl
from jax.experimental.pallas import tpu as pltpu
from jax.experimental.pallas import tpu_sc as plsc

_F = 26
_V = 100000
_D = 16
_B = 4096
_NP = _F * (_F - 1) // 2

_IIN = np.array([i for i in range(_F) for _ in range(i)], dtype=np.int32)
_JJN = np.array([j for i in range(_F) for j in range(i)], dtype=np.int32)
_SI = np.zeros((_NP, _F), np.float32)
_SI[np.arange(_NP), _IIN] = 1.0
_SJ = np.zeros((_NP, _F), np.float32)
_SJ[np.arange(_NP), _JJN] = 1.0

_NC, _NS = 2, 16
_NW = _NC * _NS
_BPW = _B // _NW
_RPW = _BPW * _F
_CW = _BPW * _D


_PPW = (_F * _D) // _NW


def _sc_gather_t(tables_t, x_t):
    mesh = plsc.VectorSubcoreMesh(
        core_axis_name="c", subcore_axis_name="s",
        num_cores=_NC, num_subcores=_NS)

    @functools.partial(
        pl.kernel,
        out_type=jax.ShapeDtypeStruct((_F, _NW, _D, _BPW), jnp.float32),
        mesh=mesh,
        scratch_types=[
            pltpu.VMEM((_V,), jnp.float32),
            pltpu.VMEM((_B,), jnp.int32),
            pltpu.VMEM((_B,), jnp.float32),
            pltpu.SemaphoreType.DMA,
        ],
        compiler_params=pltpu.CompilerParams(use_tc_tiling_on_sc=True,
                                             needs_layout_passes=False),
    )
    def gather_k(table_hbm, x_hbm, out_hbm, row_v, xrow_v, gbuf, sem):
        wid = lax.axis_index("s") * _NC + lax.axis_index("c")

        def pair_body(i, carry):
            p = wid * _PPW + i
            f = p >> 4
            d = p & (_D - 1)
            pltpu.sync_copy(table_hbm.at[f, d, :], row_v)
            pltpu.sync_copy(x_hbm.at[f, :], xrow_v)

            def g_body(c, c2):
                xi = xrow_v[pl.ds(c * _D, _D)]
                gbuf[pl.ds(c * _D, _D)] = plsc.load_gather(row_v, [xi])
                return c2

            lax.fori_loop(0, _B // _D, g_body, 0)

            def w_body(c, c2):
                pltpu.sync_copy(gbuf.at[pl.ds(c * _BPW, _BPW)],
                                out_hbm.at[f, c, d, :])
                return c2

            lax.fori_loop(0, _NW, w_body, 0)
            return carry

        lax.fori_loop(0, _PPW, pair_body, 0)

    return gather_k(tables_t, x_t)


_CH = 4
_COLS = _CH * _CW
_HI = dict(preferred_element_type=jnp.float32, precision=lax.Precision.HIGHEST)
_HX = dict(preferred_element_type=jnp.float32, precision=lax.Precision.HIGHEST)


def _tc_body(si_ref, sj_ref, sit_ref, sjt_ref, awt_ref, flag_ref, bias_ref,
             e_ref, out_ref):
    si = si_ref[...]
    sj = sj_ref[...]
    sit = sit_ref[...]
    sjt = sjt_ref[...]
    awt = awt_ref[...]
    flag = flag_ref[0]
    mx = jnp.max(awt, axis=0, keepdims=True)
    iota = lax.broadcasted_iota(jnp.int32, (5, _NP), 0)
    first = jnp.min(jnp.where(awt == mx, iota, 2**30), axis=0, keepdims=True)
    onehot = (iota == first).astype(jnp.float32)
    w = jnp.where(flag == 0, onehot, awt)
    car = w[0:1] + w[4:5] + 0.5 * (w[2:3] + w[3:4])
    cbr = w[1:2]
    ccr = 0.5 * (w[2:3] - w[3:4])

    eall = e_ref[...]
    dmat = jnp.dot(si - sj, eall, **_HI)
    s_abs = jnp.dot(ccr, jnp.abs(dmat), **_HI)
    bmat = (jnp.dot(sit * cbr, sj, **_HI)
            + jnp.dot(sjt * cbr, si, **_HI))
    m = jnp.dot(bmat, eall, **_HI)
    s_mult = 0.5 * jnp.sum(m * eall, axis=0, keepdims=True)
    arow = jnp.dot(car, si + sj, **_HX)
    s_lin = jnp.dot(arow, eall, **_HX)
    s = s_abs + s_mult + s_lin

    pieces = []
    for c in range(_CH):
        acc = s[:, c * _CW:c * _CW + _BPW]
        for d in range(1, _D):
            acc = acc + s[:, c * _CW + d * _BPW:c * _CW + (d + 1) * _BPW]
        pieces.append(acc)
    tot = jnp.concatenate(pieces, axis=1) + bias_ref[0]
    out_ref[...] = 1.0 / (1.0 + jnp.exp(-tot))


def kernel(x, flag, tables, arch_weights, bias):
    x = x.astype(jnp.int32)
    tables_t = jnp.transpose(tables, (0, 2, 1))
    e4 = _sc_gather_t(tables_t, x.T)
    e2 = e4.reshape(_F, _NW * _CW)
    flag_arr = jnp.asarray(flag, jnp.int32).reshape(1)
    out2d = pl.pallas_call(
        _tc_body,
        grid=(_NW // _CH,),
        in_specs=[
            pl.BlockSpec((_NP, _F), lambda i: (0, 0)),
            pl.BlockSpec((_NP, _F), lambda i: (0, 0)),
            pl.BlockSpec((_F, _NP), lambda i: (0, 0)),
            pl.BlockSpec((_F, _NP), lambda i: (0, 0)),
            pl.BlockSpec((5, _NP), lambda i: (0, 0)),
            pl.BlockSpec(memory_space=pltpu.SMEM),
            pl.BlockSpec(memory_space=pltpu.SMEM),
            pl.BlockSpec((_F, _COLS), lambda i: (0, i)),
        ],
        out_specs=pl.BlockSpec((1, _CH * _BPW), lambda i: (0, i)),
        out_shape=jax.ShapeDtypeStruct((1, _B), jnp.float32),
    )(jnp.asarray(_SI), jnp.asarray(_SJ), jnp.asarray(_SI.T), jnp.asarray(_SJ.T),
      arch_weights.T, flag_arr, bias, e2)
    return out2d.reshape(_B)


# speedup vs baseline: 3.9758x; 1.2548x over previous
import functools

import numpy as np
import jax
import jax.numpy as jnp
from jax import lax
from jax.experimental import pallas as pl
from jax.experimental.pallas import tpu as pltpu
from jax.experimental.pallas import tpu_sc as plsc

_F = 26
_V = 100000
_D = 16
_B = 4096
_NP = _F * (_F - 1) // 2

_IIN = np.array([i for i in range(_F) for _ in range(i)], dtype=np.int32)
_JJN = np.array([j for i in range(_F) for j in range(i)], dtype=np.int32)
_SI = np.zeros((_NP, _F), np.float32)
_SI[np.arange(_NP), _IIN] = 1.0
_SJ = np.zeros((_NP, _F), np.float32)
_SJ[np.arange(_NP), _JJN] = 1.0

_NC, _NS = 2, 16
_NW = _NC * _NS
_BPW = _B // _NW
_RPW = _BPW * _F
_CW = _BPW * _D


_PPW = (_F * _D) // _NW


def _sc_gather_t(tables_t, x_t):
    mesh = plsc.VectorSubcoreMesh(
        core_axis_name="c", subcore_axis_name="s",
        num_cores=_NC, num_subcores=_NS)

    @functools.partial(
        pl.kernel,
        out_type=jax.ShapeDtypeStruct((_F, _NW, _D, _BPW), jnp.float32),
        mesh=mesh,
        scratch_types=[
            pltpu.VMEM((_V,), jnp.float32),
            pltpu.VMEM((_B,), jnp.int32),
            pltpu.VMEM((_NW, _BPW), jnp.float32),
            pltpu.SemaphoreType.DMA,
        ],
        compiler_params=pltpu.CompilerParams(use_tc_tiling_on_sc=True,
                                             needs_layout_passes=False),
    )
    def gather_k(table_hbm, x_hbm, out_hbm, row_v, xrow_v, gbuf, sem):
        wid = lax.axis_index("s") * _NC + lax.axis_index("c")

        def pair_body(i, carry):
            p = wid * _PPW + i
            f = p >> 4
            d = p & (_D - 1)
            pltpu.sync_copy(table_hbm.at[f, d, :], row_v)
            pltpu.sync_copy(x_hbm.at[f, :], xrow_v)

            def g_body(c, c2):
                xi = xrow_v[pl.ds(c * _D, _D)]
                gbuf[c >> 3, pl.ds((c & 7) * _D, _D)] = (
                    plsc.load_gather(row_v, [xi]))
                return c2

            lax.fori_loop(0, _B // _D, g_body, 0)
            pltpu.sync_copy(gbuf, out_hbm.at[f, :, d, :])
            return carry

        lax.fori_loop(0, _PPW, pair_body, 0)

    return gather_k(tables_t, x_t)


_CH = 4
_COLS = _CH * _CW
_HI = dict(preferred_element_type=jnp.float32, precision=lax.Precision.HIGHEST)
_HX = dict(preferred_element_type=jnp.float32, precision=lax.Precision.HIGHEST)


def _tc_body(si_ref, sj_ref, sit_ref, sjt_ref, awt_ref, flag_ref, bias_ref,
             e_ref, out_ref):
    si = si_ref[...]
    sj = sj_ref[...]
    sit = sit_ref[...]
    sjt = sjt_ref[...]
    awt = awt_ref[...]
    flag = flag_ref[0]
    mx = jnp.max(awt, axis=0, keepdims=True)
    iota = lax.broadcasted_iota(jnp.int32, (5, _NP), 0)
    first = jnp.min(jnp.where(awt == mx, iota, 2**30), axis=0, keepdims=True)
    onehot = (iota == first).astype(jnp.float32)
    w = jnp.where(flag == 0, onehot, awt)
    car = w[0:1] + w[4:5] + 0.5 * (w[2:3] + w[3:4])
    cbr = w[1:2]
    ccr = 0.5 * (w[2:3] - w[3:4])

    eall = e_ref[...]
    e_hi = eall.astype(jnp.bfloat16)
    e_lo = (eall - e_hi.astype(jnp.float32)).astype(jnp.bfloat16)
    sd = (si - sj).astype(jnp.bfloat16)
    dmat = (jnp.dot(sd, e_hi, preferred_element_type=jnp.float32)
            + jnp.dot(sd, e_lo, preferred_element_type=jnp.float32))
    s_abs = jnp.dot(ccr, jnp.abs(dmat), **_HI)
    bmat = (jnp.dot(sit * cbr, sj, **_HI)
            + jnp.dot(sjt * cbr, si, **_HI))
    b_hi = bmat.astype(jnp.bfloat16)
    b_lo = (bmat - b_hi.astype(jnp.float32)).astype(jnp.bfloat16)
    m = (jnp.dot(b_hi, e_hi, preferred_element_type=jnp.float32)
         + jnp.dot(b_hi, e_lo, preferred_element_type=jnp.float32)
         + jnp.dot(b_lo, e_hi, preferred_element_type=jnp.float32))
    s_mult = 0.5 * jnp.sum(m * eall, axis=0, keepdims=True)
    arow = jnp.dot(car, si + sj, **_HX)
    s_lin = jnp.dot(arow, eall, **_HX)
    s = s_abs + s_mult + s_lin

    pieces = []
    for c in range(_CH):
        acc = s[:, c * _CW:c * _CW + _BPW]
        for d in range(1, _D):
            acc = acc + s[:, c * _CW + d * _BPW:c * _CW + (d + 1) * _BPW]
        pieces.append(acc)
    tot = jnp.concatenate(pieces, axis=1) + bias_ref[0]
    out_ref[...] = 1.0 / (1.0 + jnp.exp(-tot))


def kernel(x, flag, tables, arch_weights, bias):
    x = x.astype(jnp.int32)
    tables_t = jnp.transpose(tables, (0, 2, 1))
    e4 = _sc_gather_t(tables_t, x.T)
    e2 = e4.reshape(_F, _NW * _CW)
    flag_arr = jnp.asarray(flag, jnp.int32).reshape(1)
    out2d = pl.pallas_call(
        _tc_body,
        grid=(_NW // _CH,),
        in_specs=[
            pl.BlockSpec((_NP, _F), lambda i: (0, 0)),
            pl.BlockSpec((_NP, _F), lambda i: (0, 0)),
            pl.BlockSpec((_F, _NP), lambda i: (0, 0)),
            pl.BlockSpec((_F, _NP), lambda i: (0, 0)),
            pl.BlockSpec((5, _NP), lambda i: (0, 0)),
            pl.BlockSpec(memory_space=pltpu.SMEM),
            pl.BlockSpec(memory_space=pltpu.SMEM),
            pl.BlockSpec((_F, _COLS), lambda i: (0, i)),
        ],
        out_specs=pl.BlockSpec((1, _CH * _BPW), lambda i: (0, i)),
        out_shape=jax.ShapeDtypeStruct((1, _B), jnp.float32),
    )(jnp.asarray(_SI), jnp.asarray(_SJ), jnp.asarray(_SI.T), jnp.asarray(_SJ.T),
      arch_weights.T, flag_arr, bias, e2)
    return out2d.reshape(_B)


# speedup vs baseline: 4.8388x; 1.2171x over previous
import functools

import numpy as np
import jax
import jax.numpy as jnp
from jax import lax
from jax.experimental import pallas as pl
from jax.experimental.pallas import tpu as pltpu
from jax.experimental.pallas import tpu_sc as plsc

_F = 26
_V = 100000
_D = 16
_B = 4096
_NP = _F * (_F - 1) // 2

_IIN = np.array([i for i in range(_F) for _ in range(i)], dtype=np.int32)
_JJN = np.array([j for i in range(_F) for j in range(i)], dtype=np.int32)
_SI = np.zeros((_NP, _F), np.float32)
_SI[np.arange(_NP), _IIN] = 1.0
_SJ = np.zeros((_NP, _F), np.float32)
_SJ[np.arange(_NP), _JJN] = 1.0

_NC, _NS = 2, 16
_NW = _NC * _NS
_BPW = _B // _NW
_RPW = _BPW * _F
_CW = _BPW * _D


_PPW = (_F * _D) // _NW


def _sc_gather_t(tables_t, x_t):
    mesh = plsc.VectorSubcoreMesh(
        core_axis_name="c", subcore_axis_name="s",
        num_cores=_NC, num_subcores=_NS)

    @functools.partial(
        pl.kernel,
        out_type=jax.ShapeDtypeStruct((_F, _NW, _D, _BPW), jnp.float32),
        mesh=mesh,
        scratch_types=[
            pltpu.VMEM((_V,), jnp.float32),
            pltpu.VMEM((_B,), jnp.int32),
            pltpu.VMEM((_NW, _BPW), jnp.float32),
            pltpu.SemaphoreType.DMA,
        ],
        compiler_params=pltpu.CompilerParams(use_tc_tiling_on_sc=True,
                                             needs_layout_passes=False),
    )
    def gather_k(table_hbm, x_hbm, out_hbm, row_v, xrow_v, gbuf, sem):
        wid = lax.axis_index("s") * _NC + lax.axis_index("c")

        def pair_body(i, carry):
            p = wid * _PPW + i
            f = p >> 4
            d = p & (_D - 1)
            pltpu.sync_copy(table_hbm.at[f, d, :], row_v)
            pltpu.sync_copy(x_hbm.at[f, :], xrow_v)

            def g_body(c, c2):
                xi = xrow_v[pl.ds(c * _D, _D)]
                gbuf[c >> 3, pl.ds((c & 7) * _D, _D)] = (
                    plsc.load_gather(row_v, [xi]))
                return c2

            lax.fori_loop(0, _B // _D, g_body, 0)
            pltpu.sync_copy(gbuf, out_hbm.at[f, :, d, :])
            return carry

        lax.fori_loop(0, _PPW, pair_body, 0)

    return gather_k(tables_t, x_t)


_CH = 4
_COLS = _CH * _CW
_HI = dict(preferred_element_type=jnp.float32, precision=lax.Precision.HIGHEST)
_HX = dict(preferred_element_type=jnp.float32, precision=lax.Precision.HIGHEST)


def _tc_body(si_ref, sj_ref, sit_ref, sjt_ref, awt_ref, flag_ref, bias_ref,
             e_ref, out_ref):
    si = si_ref[...]
    sj = sj_ref[...]
    sit = sit_ref[...]
    sjt = sjt_ref[...]
    awt = awt_ref[...]
    flag = flag_ref[0]
    mx = jnp.max(awt, axis=0, keepdims=True)
    iota = lax.broadcasted_iota(jnp.int32, (5, _NP), 0)
    first = jnp.min(jnp.where(awt == mx, iota, 2**30), axis=0, keepdims=True)
    onehot = (iota == first).astype(jnp.float32)
    w = jnp.where(flag == 0, onehot, awt)
    car = w[0:1] + w[4:5] + 0.5 * (w[2:3] + w[3:4])
    cbr = w[1:2]
    ccr = 0.5 * (w[2:3] - w[3:4])

    eall = e_ref[...]
    e_hi = eall.astype(jnp.bfloat16)
    e_lo = (eall - e_hi.astype(jnp.float32)).astype(jnp.bfloat16)
    sd = (si - sj).astype(jnp.bfloat16)
    dmat = (jnp.dot(sd, e_hi, preferred_element_type=jnp.float32)
            + jnp.dot(sd, e_lo, preferred_element_type=jnp.float32))
    s_abs = jnp.dot(ccr.astype(jnp.bfloat16),
                    jnp.abs(dmat).astype(jnp.bfloat16),
                    preferred_element_type=jnp.float32)
    bmat = (jnp.dot(sit * cbr, sj, **_HI)
            + jnp.dot(sjt * cbr, si, **_HI))
    b_hi = bmat.astype(jnp.bfloat16)
    b_lo = (bmat - b_hi.astype(jnp.float32)).astype(jnp.bfloat16)
    m = (jnp.dot(b_hi, e_hi, preferred_element_type=jnp.float32)
         + jnp.dot(b_hi, e_lo, preferred_element_type=jnp.float32)
         + jnp.dot(b_lo, e_hi, preferred_element_type=jnp.float32))
    s_mult = 0.5 * jnp.sum(m * eall, axis=0, keepdims=True)
    arow = jnp.dot(car, si + sj, **_HX)
    a_hi = arow.astype(jnp.bfloat16)
    a_lo = (arow - a_hi.astype(jnp.float32)).astype(jnp.bfloat16)
    s_lin = (jnp.dot(a_hi, e_hi, preferred_element_type=jnp.float32)
             + jnp.dot(a_hi, e_lo, preferred_element_type=jnp.float32)
             + jnp.dot(a_lo, e_hi, preferred_element_type=jnp.float32))
    s = s_abs + s_mult + s_lin

    pieces = []
    for c in range(_CH):
        acc = s[:, c * _CW:c * _CW + _BPW]
        for d in range(1, _D):
            acc = acc + s[:, c * _CW + d * _BPW:c * _CW + (d + 1) * _BPW]
        pieces.append(acc)
    tot = jnp.concatenate(pieces, axis=1) + bias_ref[0]
    out_ref[...] = 1.0 / (1.0 + jnp.exp(-tot))


def kernel(x, flag, tables, arch_weights, bias):
    x = x.astype(jnp.int32)
    tables_t = jnp.transpose(tables, (0, 2, 1))
    e4 = _sc_gather_t(tables_t, x.T)
    e2 = e4.reshape(_F, _NW * _CW)
    flag_arr = jnp.asarray(flag, jnp.int32).reshape(1)
    out2d = pl.pallas_call(
        _tc_body,
        grid=(_NW // _CH,),
        in_specs=[
            pl.BlockSpec((_NP, _F), lambda i: (0, 0)),
            pl.BlockSpec((_NP, _F), lambda i: (0, 0)),
            pl.BlockSpec((_F, _NP), lambda i: (0, 0)),
            pl.BlockSpec((_F, _NP), lambda i: (0, 0)),
            pl.BlockSpec((5, _NP), lambda i: (0, 0)),
            pl.BlockSpec(memory_space=pltpu.SMEM),
            pl.BlockSpec(memory_space=pltpu.SMEM),
            pl.BlockSpec((_F, _COLS), lambda i: (0, i)),
        ],
        out_specs=pl.BlockSpec((1, _CH * _BPW), lambda i: (0, i)),
        out_shape=jax.ShapeDtypeStruct((1, _B), jnp.float32),
    )(jnp.asarray(_SI), jnp.asarray(_SJ), jnp.asarray(_SI.T), jnp.asarray(_SJ.T),
      arch_weights.T, flag_arr, bias, e2)
    return out2d.reshape(_B)


# speedup vs baseline: 4.9597x; 1.0250x over previous
import functools

import numpy as np
import jax
import jax.numpy as jnp
from jax import lax
from jax.experimental import pallas as pl
from jax.experimental.pallas import tpu as pltpu
from jax.experimental.pallas import tpu_sc as plsc

_F = 26
_V = 100000
_D = 16
_B = 4096
_NP = _F * (_F - 1) // 2

_IIN = np.array([i for i in range(_F) for _ in range(i)], dtype=np.int32)
_JJN = np.array([j for i in range(_F) for j in range(i)], dtype=np.int32)
_SI = np.zeros((_NP, _F), np.float32)
_SI[np.arange(_NP), _IIN] = 1.0
_SJ = np.zeros((_NP, _F), np.float32)
_SJ[np.arange(_NP), _JJN] = 1.0

_NC, _NS = 2, 16
_NW = _NC * _NS
_BPW = _B // _NW
_RPW = _BPW * _F
_CW = _BPW * _D


_PPW = (_F * _D) // _NW


def _sc_gather_t(tables_t, x_t):
    mesh = plsc.VectorSubcoreMesh(
        core_axis_name="c", subcore_axis_name="s",
        num_cores=_NC, num_subcores=_NS)

    @functools.partial(
        pl.kernel,
        out_type=jax.ShapeDtypeStruct((_F, _NW, _D, _BPW), jnp.float32),
        mesh=mesh,
        scratch_types=[
            pltpu.VMEM((_V,), jnp.float32),
            pltpu.VMEM((_B,), jnp.int32),
            pltpu.VMEM((_NW, _BPW), jnp.float32),
            pltpu.SemaphoreType.DMA,
        ],
        compiler_params=pltpu.CompilerParams(use_tc_tiling_on_sc=True,
                                             needs_layout_passes=False),
    )
    def gather_k(table_hbm, x_hbm, out_hbm, row_v, xrow_v, gbuf, sem):
        wid = lax.axis_index("s") * _NC + lax.axis_index("c")

        def pair_body(i, carry):
            p = wid * _PPW + i
            f = p >> 4
            d = p & (_D - 1)
            pltpu.sync_copy(table_hbm.at[f, d, :], row_v)
            pltpu.sync_copy(x_hbm.at[f, :], xrow_v)

            def g_body(c, c2):
                xi = xrow_v[pl.ds(c * _D, _D)]
                gbuf[c >> 3, pl.ds((c & 7) * _D, _D)] = (
                    plsc.load_gather(row_v, [xi]))
                return c2

            lax.fori_loop(0, _B // _D, g_body, 0)
            pltpu.sync_copy(gbuf, out_hbm.at[f, :, d, :])
            return carry

        lax.fori_loop(0, _PPW, pair_body, 0)

    return gather_k(tables_t, x_t)


_CH = 8
_COLS = _CH * _CW
_HI = dict(preferred_element_type=jnp.float32, precision=lax.Precision.HIGHEST)
_HX = dict(preferred_element_type=jnp.float32, precision=lax.Precision.HIGHEST)


def _tc_body(si_ref, sj_ref, sit_ref, sjt_ref, awt_ref, flag_ref, bias_ref,
             e_ref, out_ref):
    si = si_ref[...]
    sj = sj_ref[...]
    sit = sit_ref[...]
    sjt = sjt_ref[...]
    awt = awt_ref[...]
    flag = flag_ref[0]
    mx = jnp.max(awt, axis=0, keepdims=True)
    iota = lax.broadcasted_iota(jnp.int32, (5, _NP), 0)
    first = jnp.min(jnp.where(awt == mx, iota, 2**30), axis=0, keepdims=True)
    onehot = (iota == first).astype(jnp.float32)
    w = jnp.where(flag == 0, onehot, awt)
    car = w[0:1] + w[4:5] + 0.5 * (w[2:3] + w[3:4])
    cbr = w[1:2]
    ccr = 0.5 * (w[2:3] - w[3:4])

    eall = e_ref[...]
    e_hi = eall.astype(jnp.bfloat16)
    e_lo = (eall - e_hi.astype(jnp.float32)).astype(jnp.bfloat16)
    sd = (si - sj).astype(jnp.bfloat16)
    dmat = (jnp.dot(sd, e_hi, preferred_element_type=jnp.float32)
            + jnp.dot(sd, e_lo, preferred_element_type=jnp.float32))
    s_abs = jnp.dot(ccr.astype(jnp.bfloat16),
                    jnp.abs(dmat).astype(jnp.bfloat16),
                    preferred_element_type=jnp.float32)
    bmat = (jnp.dot(sit * cbr, sj, **_HI)
            + jnp.dot(sjt * cbr, si, **_HI))
    b_hi = bmat.astype(jnp.bfloat16)
    b_lo = (bmat - b_hi.astype(jnp.float32)).astype(jnp.bfloat16)
    m = (jnp.dot(b_hi, e_hi, preferred_element_type=jnp.float32)
         + jnp.dot(b_hi, e_lo, preferred_element_type=jnp.float32)
         + jnp.dot(b_lo, e_hi, preferred_element_type=jnp.float32))
    s_mult = 0.5 * jnp.sum(m * eall, axis=0, keepdims=True)
    arow = jnp.dot(car, si + sj, **_HX)
    a_hi = arow.astype(jnp.bfloat16)
    a_lo = (arow - a_hi.astype(jnp.float32)).astype(jnp.bfloat16)
    s_lin = (jnp.dot(a_hi, e_hi, preferred_element_type=jnp.float32)
             + jnp.dot(a_hi, e_lo, preferred_element_type=jnp.float32)
             + jnp.dot(a_lo, e_hi, preferred_element_type=jnp.float32))
    s = s_abs + s_mult + s_lin

    pieces = []
    for c in range(_CH):
        acc = s[:, c * _CW:c * _CW + _BPW]
        for d in range(1, _D):
            acc = acc + s[:, c * _CW + d * _BPW:c * _CW + (d + 1) * _BPW]
        pieces.append(acc)
    tot = jnp.concatenate(pieces, axis=1) + bias_ref[0]
    out_ref[...] = 1.0 / (1.0 + jnp.exp(-tot))


def kernel(x, flag, tables, arch_weights, bias):
    x = x.astype(jnp.int32)
    tables_t = jnp.transpose(tables, (0, 2, 1))
    e4 = _sc_gather_t(tables_t, x.T)
    e2 = e4.reshape(_F, _NW * _CW)
    flag_arr = jnp.asarray(flag, jnp.int32).reshape(1)
    out2d = pl.pallas_call(
        _tc_body,
        grid=(_NW // _CH,),
        in_specs=[
            pl.BlockSpec((_NP, _F), lambda i: (0, 0)),
            pl.BlockSpec((_NP, _F), lambda i: (0, 0)),
            pl.BlockSpec((_F, _NP), lambda i: (0, 0)),
            pl.BlockSpec((_F, _NP), lambda i: (0, 0)),
            pl.BlockSpec((5, _NP), lambda i: (0, 0)),
            pl.BlockSpec(memory_space=pltpu.SMEM),
            pl.BlockSpec(memory_space=pltpu.SMEM),
            pl.BlockSpec((_F, _COLS), lambda i: (0, i)),
        ],
        out_specs=pl.BlockSpec((1, _CH * _BPW), lambda i: (0, i)),
        out_shape=jax.ShapeDtypeStruct((1, _B), jnp.float32),
    )(jnp.asarray(_SI), jnp.asarray(_SJ), jnp.asarray(_SI.T), jnp.asarray(_SJ.T),
      arch_weights.T, flag_arr, bias, e2)
    return out2d.reshape(_B)


# speedup vs baseline: 5.1702x; 1.0424x over previous
import functools

import numpy as np
import jax
import jax.numpy as jnp
from jax import lax
from jax.experimental import pallas as pl
from jax.experimental.pallas import tpu as pltpu
from jax.experimental.pallas import tpu_sc as plsc

_F = 26
_V = 100000
_D = 16
_B = 4096
_NP = _F * (_F - 1) // 2

_IIN = np.array([i for i in range(_F) for _ in range(i)], dtype=np.int32)
_JJN = np.array([j for i in range(_F) for j in range(i)], dtype=np.int32)
_SI = np.zeros((_NP, _F), np.float32)
_SI[np.arange(_NP), _IIN] = 1.0
_SJ = np.zeros((_NP, _F), np.float32)
_SJ[np.arange(_NP), _JJN] = 1.0

_NC, _NS = 2, 16
_NW = _NC * _NS
_BPW = _B // _NW
_RPW = _BPW * _F
_CW = _BPW * _D


_PPW = (_F * _D) // _NW


def _sc_gather_t(tables_t, x_t):
    mesh = plsc.VectorSubcoreMesh(
        core_axis_name="c", subcore_axis_name="s",
        num_cores=_NC, num_subcores=_NS)

    @functools.partial(
        pl.kernel,
        out_type=jax.ShapeDtypeStruct((_F, _NW, _D, _BPW), jnp.float32),
        mesh=mesh,
        scratch_types=[
            pltpu.VMEM((_V,), jnp.float32),
            pltpu.VMEM((_B,), jnp.int32),
            pltpu.VMEM((_NW, _BPW), jnp.float32),
            pltpu.SemaphoreType.DMA,
        ],
        compiler_params=pltpu.CompilerParams(use_tc_tiling_on_sc=True,
                                             needs_layout_passes=False),
    )
    def gather_k(table_hbm, x_hbm, out_hbm, row_v, xrow_v, gbuf, sem):
        wid = lax.axis_index("s") * _NC + lax.axis_index("c")

        def pair_body(i, carry):
            p = wid * _PPW + i
            f = p >> 4
            d = p & (_D - 1)
            cp = pltpu.async_copy(table_hbm.at[f, d, :], row_v, sem)
            pltpu.sync_copy(x_hbm.at[f, :], xrow_v)
            cp.wait()

            def g_body(c, c2):
                xi = xrow_v[pl.ds(c * _D, _D)]
                gbuf[c >> 3, pl.ds((c & 7) * _D, _D)] = (
                    plsc.load_gather(row_v, [xi]))
                return c2

            lax.fori_loop(0, _B // _D, g_body, 0)
            pltpu.sync_copy(gbuf, out_hbm.at[f, :, d, :])
            return carry

        lax.fori_loop(0, _PPW, pair_body, 0)

    return gather_k(tables_t, x_t)


_CH = 8
_COLS = _CH * _CW
_HI = dict(preferred_element_type=jnp.float32, precision=lax.Precision.HIGHEST)
_HX = dict(preferred_element_type=jnp.float32, precision=lax.Precision.HIGHEST)


def _tc_body(si_ref, sj_ref, sit_ref, sjt_ref, awt_ref, flag_ref, bias_ref,
             e_ref, out_ref):
    si = si_ref[...]
    sj = sj_ref[...]
    sit = sit_ref[...]
    sjt = sjt_ref[...]
    awt = awt_ref[...]
    flag = flag_ref[0]
    mx = jnp.max(awt, axis=0, keepdims=True)
    iota = lax.broadcasted_iota(jnp.int32, (5, _NP), 0)
    first = jnp.min(jnp.where(awt == mx, iota, 2**30), axis=0, keepdims=True)
    onehot = (iota == first).astype(jnp.float32)
    w = jnp.where(flag == 0, onehot, awt)
    car = w[0:1] + w[4:5] + 0.5 * (w[2:3] + w[3:4])
    cbr = w[1:2]
    ccr = 0.5 * (w[2:3] - w[3:4])

    eall = e_ref[...]
    e_hi = eall.astype(jnp.bfloat16)
    e_lo = (eall - e_hi.astype(jnp.float32)).astype(jnp.bfloat16)
    sd = (si - sj).astype(jnp.bfloat16)
    dmat = (jnp.dot(sd, e_hi, preferred_element_type=jnp.float32)
            + jnp.dot(sd, e_lo, preferred_element_type=jnp.float32))
    s_abs = jnp.dot(ccr.astype(jnp.bfloat16),
                    jnp.abs(dmat).astype(jnp.bfloat16),
                    preferred_element_type=jnp.float32)
    bmat = (jnp.dot(sit * cbr, sj, **_HI)
            + jnp.dot(sjt * cbr, si, **_HI))
    b_hi = bmat.astype(jnp.bfloat16)
    b_lo = (bmat - b_hi.astype(jnp.float32)).astype(jnp.bfloat16)
    m = (jnp.dot(b_hi, e_hi, preferred_element_type=jnp.float32)
         + jnp.dot(b_hi, e_lo, preferred_element_type=jnp.float32)
         + jnp.dot(b_lo, e_hi, preferred_element_type=jnp.float32))
    s_mult = 0.5 * jnp.sum(m * eall, axis=0, keepdims=True)
    arow = jnp.dot(car, si + sj, **_HX)
    a_hi = arow.astype(jnp.bfloat16)
    a_lo = (arow - a_hi.astype(jnp.float32)).astype(jnp.bfloat16)
    s_lin = (jnp.dot(a_hi, e_hi, preferred_element_type=jnp.float32)
             + jnp.dot(a_hi, e_lo, preferred_element_type=jnp.float32)
             + jnp.dot(a_lo, e_hi, preferred_element_type=jnp.float32))
    s = s_abs + s_mult + s_lin

    pieces = []
    for c in range(_CH):
        acc = s[:, c * _CW:c * _CW + _BPW]
        for d in range(1, _D):
            acc = acc + s[:, c * _CW + d * _BPW:c * _CW + (d + 1) * _BPW]
        pieces.append(acc)
    tot = jnp.concatenate(pieces, axis=1) + bias_ref[0]
    out_ref[...] = 1.0 / (1.0 + jnp.exp(-tot))


def kernel(x, flag, tables, arch_weights, bias):
    x = x.astype(jnp.int32)
    tables_t = jnp.transpose(tables, (0, 2, 1))
    e4 = _sc_gather_t(tables_t, x.T)
    e2 = e4.reshape(_F, _NW * _CW)
    flag_arr = jnp.asarray(flag, jnp.int32).reshape(1)
    out2d = pl.pallas_call(
        _tc_body,
        grid=(_NW // _CH,),
        in_specs=[
            pl.BlockSpec((_NP, _F), lambda i: (0, 0)),
            pl.BlockSpec((_NP, _F), lambda i: (0, 0)),
            pl.BlockSpec((_F, _NP), lambda i: (0, 0)),
            pl.BlockSpec((_F, _NP), lambda i: (0, 0)),
            pl.BlockSpec((5, _NP), lambda i: (0, 0)),
            pl.BlockSpec(memory_space=pltpu.SMEM),
            pl.BlockSpec(memory_space=pltpu.SMEM),
            pl.BlockSpec((_F, _COLS), lambda i: (0, i)),
        ],
        out_specs=pl.BlockSpec((1, _CH * _BPW), lambda i: (0, i)),
        out_shape=jax.ShapeDtypeStruct((1, _B), jnp.float32),
    )(jnp.asarray(_SI), jnp.asarray(_SJ), jnp.asarray(_SI.T), jnp.asarray(_SJ.T),
      arch_weights.T, flag_arr, bias, e2)
    return out2d.reshape(_B)


# speedup vs baseline: 5.2017x; 1.0061x over previous
import functools

import numpy as np
import jax
import jax.numpy as jnp
from jax import lax
from jax.experimental import pallas as pl
from jax.experimental.pallas import tpu as pltpu
from jax.experimental.pallas import tpu_sc as plsc

_F = 26
_V = 100000
_D = 16
_B = 4096
_NP = _F * (_F - 1) // 2

_IIN = np.array([i for i in range(_F) for _ in range(i)], dtype=np.int32)
_JJN = np.array([j for i in range(_F) for j in range(i)], dtype=np.int32)
_SI = np.zeros((_NP, _F), np.float32)
_SI[np.arange(_NP), _IIN] = 1.0
_SJ = np.zeros((_NP, _F), np.float32)
_SJ[np.arange(_NP), _JJN] = 1.0

_NC, _NS = 2, 16
_NW = _NC * _NS
_BPW = _B // _NW
_CW = _BPW * _D


_PPW = (_F * _D) // _NW


def _sc_gather_t(tables_t, x_t):
    mesh = plsc.VectorSubcoreMesh(
        core_axis_name="c", subcore_axis_name="s",
        num_cores=_NC, num_subcores=_NS)

    @functools.partial(
        pl.kernel,
        out_type=jax.ShapeDtypeStruct((_F, _NW, _D, _BPW), jnp.float32),
        mesh=mesh,
        scratch_types=[
            pltpu.VMEM((_V,), jnp.float32),
            pltpu.VMEM((_B,), jnp.int32),
            pltpu.VMEM((_NW, _BPW), jnp.float32),
            pltpu.SemaphoreType.DMA,
        ],
        compiler_params=pltpu.CompilerParams(use_tc_tiling_on_sc=True,
                                             needs_layout_passes=False),
    )
    def gather_k(table_hbm, x_hbm, out_hbm, row_v, xrow_v, gbuf, sem):
        wid = lax.axis_index("s") * _NC + lax.axis_index("c")

        def pair_body(i, carry):
            p = wid * _PPW + i
            f = p >> 4
            d = p & (_D - 1)
            cp = pltpu.async_copy(table_hbm.at[f, d, :], row_v, sem)
            pltpu.sync_copy(x_hbm.at[f, :], xrow_v)
            cp.wait()

            def g_body(c, c2):
                xi = xrow_v[pl.ds(c * _D, _D)]
                gbuf[c >> 3, pl.ds((c & 7) * _D, _D)] = (
                    plsc.load_gather(row_v, [xi]))
                return c2

            lax.fori_loop(0, _B // _D, g_body, 0)
            pltpu.sync_copy(gbuf, out_hbm.at[f, :, d, :])
            return carry

        lax.fori_loop(0, _PPW, pair_body, 0)

    return gather_k(tables_t, x_t)


_CH = 16
_COLS = _CH * _CW
_HI = dict(preferred_element_type=jnp.float32, precision=lax.Precision.HIGHEST)
_HX = dict(preferred_element_type=jnp.float32, precision=lax.Precision.HIGHEST)


def _tc_body(si_ref, sj_ref, sit_ref, sjt_ref, awt_ref, flag_ref, bias_ref,
             e_ref, out_ref):
    si = si_ref[...]
    sj = sj_ref[...]
    sit = sit_ref[...]
    sjt = sjt_ref[...]
    awt = awt_ref[...]
    flag = flag_ref[0]
    mx = jnp.max(awt, axis=0, keepdims=True)
    iota = lax.broadcasted_iota(jnp.int32, (5, _NP), 0)
    first = jnp.min(jnp.where(awt == mx, iota, 2**30), axis=0, keepdims=True)
    onehot = (iota == first).astype(jnp.float32)
    w = jnp.where(flag == 0, onehot, awt)
    car = w[0:1] + w[4:5] + 0.5 * (w[2:3] + w[3:4])
    cbr = w[1:2]
    ccr = 0.5 * (w[2:3] - w[3:4])

    eall = e_ref[...]
    e_hi = eall.astype(jnp.bfloat16)
    e_lo = (eall - e_hi.astype(jnp.float32)).astype(jnp.bfloat16)
    sd = (si - sj).astype(jnp.bfloat16)
    dmat = (jnp.dot(sd, e_hi, preferred_element_type=jnp.float32)
            + jnp.dot(sd, e_lo, preferred_element_type=jnp.float32))
    s_abs = jnp.dot(ccr.astype(jnp.bfloat16),
                    jnp.abs(dmat).astype(jnp.bfloat16),
                    preferred_element_type=jnp.float32)
    bmat = (jnp.dot(sit * cbr, sj, **_HI)
            + jnp.dot(sjt * cbr, si, **_HI))
    b_hi = bmat.astype(jnp.bfloat16)
    b_lo = (bmat - b_hi.astype(jnp.float32)).astype(jnp.bfloat16)
    m = (jnp.dot(b_hi, e_hi, preferred_element_type=jnp.float32)
         + jnp.dot(b_hi, e_lo, preferred_element_type=jnp.float32)
         + jnp.dot(b_lo, e_hi, preferred_element_type=jnp.float32))
    s_mult = 0.5 * jnp.sum(m * eall, axis=0, keepdims=True)
    arow = jnp.dot(car, si + sj, **_HX)
    a_hi = arow.astype(jnp.bfloat16)
    a_lo = (arow - a_hi.astype(jnp.float32)).astype(jnp.bfloat16)
    s_lin = (jnp.dot(a_hi, e_hi, preferred_element_type=jnp.float32)
             + jnp.dot(a_hi, e_lo, preferred_element_type=jnp.float32)
             + jnp.dot(a_lo, e_hi, preferred_element_type=jnp.float32))
    s = s_abs + s_mult + s_lin

    pieces = []
    for c in range(_CH):
        acc = s[:, c * _CW:c * _CW + _BPW]
        for d in range(1, _D):
            acc = acc + s[:, c * _CW + d * _BPW:c * _CW + (d + 1) * _BPW]
        pieces.append(acc)
    tot = jnp.concatenate(pieces, axis=1) + bias_ref[0]
    out_ref[...] = 1.0 / (1.0 + jnp.exp(-tot))


def kernel(x, flag, tables, arch_weights, bias):
    x = x.astype(jnp.int32)
    tables_t = jnp.transpose(tables, (0, 2, 1))
    e4 = _sc_gather_t(tables_t, x.T)
    e2 = e4.reshape(_F, _NW * _CW)
    flag_arr = jnp.asarray(flag, jnp.int32).reshape(1)
    out2d = pl.pallas_call(
        _tc_body,
        grid=(_NW // _CH,),
        in_specs=[
            pl.BlockSpec((_NP, _F), lambda i: (0, 0)),
            pl.BlockSpec((_NP, _F), lambda i: (0, 0)),
            pl.BlockSpec((_F, _NP), lambda i: (0, 0)),
            pl.BlockSpec((_F, _NP), lambda i: (0, 0)),
            pl.BlockSpec((5, _NP), lambda i: (0, 0)),
            pl.BlockSpec(memory_space=pltpu.SMEM),
            pl.BlockSpec(memory_space=pltpu.SMEM),
            pl.BlockSpec((_F, _COLS), lambda i: (0, i)),
        ],
        out_specs=pl.BlockSpec((1, _CH * _BPW), lambda i: (0, i)),
        out_shape=jax.ShapeDtypeStruct((1, _B), jnp.float32),
    )(jnp.asarray(_SI), jnp.asarray(_SJ), jnp.asarray(_SI.T), jnp.asarray(_SJ.T),
      arch_weights.T, flag_arr, bias, e2)
    return out2d.reshape(_B)
